# speedup vs baseline: 1.1483x; 1.1483x over previous
import functools

import jax
import jax.numpy as jnp
from jax import lax
from jax.experimental import pallas as pl
from jax.experimental.pallas import tpu as pltpu
from jax.experimental.pallas import tpu_sc as plsc

_NC = 2
_NS = 16
_NW = _NC * _NS
_LANES = 16

_CHUNK = 64


def _make_sc_embed(n_tokens, seq_len, hidden):
    per_w = n_tokens // _NW
    n_chunks = per_w // _CHUNK
    hgroups = hidden // _LANES

    mesh = plsc.VectorSubcoreMesh(core_axis_name="c", subcore_axis_name="s")

    @functools.partial(
        pl.kernel,
        mesh=mesh,
        out_type=jax.ShapeDtypeStruct((n_tokens, hidden), jnp.float32),
        scratch_types=[
            pltpu.VMEM((_CHUNK,), jnp.int32),
            pltpu.VMEM((_CHUNK, hidden), jnp.float32),
            pltpu.VMEM((_CHUNK, hidden), jnp.float32),
            pltpu.VMEM((hidden,), jnp.float32),
            pltpu.SemaphoreType.DMA,
        ],
    )
    def sc_embed(ids_hbm, wtab_hbm, ptab_hbm, ttab_hbm, out_hbm,
                 idx_v, rows_v, pos_v, type_v, sem):
        wid = lax.axis_index("s") * _NC + lax.axis_index("c")
        base = wid * per_w
        pltpu.sync_copy(ttab_hbm.at[0], type_v)
        for c in range(n_chunks):
            off = base + c * _CHUNK
            pltpu.sync_copy(ids_hbm.at[pl.ds(off, _CHUNK)], idx_v)
            gather = pltpu.async_copy(wtab_hbm.at[idx_v], rows_v, sem)
            s0 = lax.rem(off, seq_len)
            pltpu.sync_copy(ptab_hbm.at[pl.ds(s0, _CHUNK)], pos_v)
            gather.wait()

            def row_body(r, carry):
                for h in range(hgroups):
                    sl = pl.ds(h * _LANES, _LANES)
                    v = pos_v[r, sl] + type_v[sl]
                    plsc.addupdate(rows_v.at[r, sl], v)
                return carry

            lax.fori_loop(0, _CHUNK, row_body, 0)
            pltpu.sync_copy(rows_v, out_hbm.at[pl.ds(off, _CHUNK)])

    return sc_embed


def kernel(input_ids, word_emb, pos_emb, type_emb):
    b, s = input_ids.shape
    hidden = word_emb.shape[1]
    n_tokens = b * s
    assert n_tokens % (_NW * _CHUNK) == 0 and s % _CHUNK == 0

    ids_flat = input_ids.reshape(-1).astype(jnp.int32)
    fn = _make_sc_embed(n_tokens, s, hidden)
    out = fn(ids_flat, word_emb, pos_emb, type_emb)
    return out.reshape(b, s, hidden)


# speedup vs baseline: 1.3815x; 1.2031x over previous
import functools

import jax
import jax.numpy as jnp
from jax import lax
from jax.experimental import pallas as pl
from jax.experimental.pallas import tpu as pltpu
from jax.experimental.pallas import tpu_sc as plsc

_NC = 2
_NS = 16
_NW = _NC * _NS
_LANES = 16

_CHUNK = 32
_NROW = 3
_NPOS = 2


def _make_sc_embed(n_tokens, seq_len, hidden):
    per_w = n_tokens // _NW
    n_chunks = per_w // _CHUNK
    hgroups = hidden // _LANES

    mesh = plsc.VectorSubcoreMesh(core_axis_name="c", subcore_axis_name="s")

    @functools.partial(
        pl.kernel,
        mesh=mesh,
        out_type=jax.ShapeDtypeStruct((n_tokens, hidden), jnp.float32),
        scratch_types=(
            [pltpu.VMEM((per_w,), jnp.int32),
             pltpu.VMEM((hidden,), jnp.float32)]
            + [pltpu.VMEM((_CHUNK, hidden), jnp.float32) for _ in range(_NROW)]
            + [pltpu.VMEM((_CHUNK, hidden), jnp.float32) for _ in range(_NPOS)]
            + [pltpu.SemaphoreType.DMA for _ in range(2 * _NROW + _NPOS)]
        ),
    )
    def sc_embed(ids_hbm, wtab_hbm, ptab_hbm, ttab_hbm, out_hbm,
                 idx_v, type_v, *bufs_and_sems):
        rows = bufs_and_sems[:_NROW]
        pos = bufs_and_sems[_NROW:_NROW + _NPOS]
        gsem = bufs_and_sems[_NROW + _NPOS:2 * _NROW + _NPOS]
        osem = bufs_and_sems[2 * _NROW + _NPOS:3 * _NROW + _NPOS]
        psem = bufs_and_sems[3 * _NROW + _NPOS:]

        wid = lax.axis_index("s") * _NC + lax.axis_index("c")
        base = wid * per_w
        pltpu.sync_copy(ids_hbm.at[pl.ds(base, per_w)], idx_v)
        pltpu.sync_copy(ttab_hbm.at[0], type_v)

        def issue_gather(c):
            return pltpu.async_copy(
                wtab_hbm.at[idx_v.at[pl.ds(c * _CHUNK, _CHUNK)]],
                rows[c % _NROW], gsem[c % _NROW])

        def issue_pos(c):
            s0 = lax.rem(base + c * _CHUNK, seq_len)
            return pltpu.async_copy(
                ptab_hbm.at[pl.ds(s0, _CHUNK)], pos[c % _NPOS],
                psem[c % _NPOS])

        def issue_store(c):
            return pltpu.async_copy(
                rows[c % _NROW],
                out_hbm.at[pl.ds(base + c * _CHUNK, _CHUNK)],
                osem[c % _NROW])

        g = [None] * n_chunks
        p = [None] * n_chunks
        o = [None] * n_chunks
        for c in range(min(_NROW, n_chunks)):
            g[c] = issue_gather(c)
        for c in range(min(_NPOS, n_chunks)):
            p[c] = issue_pos(c)

        for c in range(n_chunks):
            if c >= 2:
                o[c - 2].wait()
                if c + 1 < n_chunks:
                    g[c + 1] = issue_gather(c + 1)
            g[c].wait()
            p[c].wait()
            rbuf = rows[c % _NROW]
            pbuf = pos[c % _NPOS]

            def row_body(r, carry):
                for h in range(hgroups):
                    sl = pl.ds(h * _LANES, _LANES)
                    v = pbuf[r, sl] + type_v[sl]
                    plsc.addupdate(rbuf.at[r, sl], v)
                return carry

            lax.fori_loop(0, _CHUNK, row_body, 0)
            o[c] = issue_store(c)
            if c + _NPOS < n_chunks:
                p[c + _NPOS] = issue_pos(c + _NPOS)

        for c in range(max(0, n_chunks - 2), n_chunks):
            o[c].wait()

    return sc_embed


def kernel(input_ids, word_emb, pos_emb, type_emb):
    b, s = input_ids.shape
    hidden = word_emb.shape[1]
    n_tokens = b * s
    assert n_tokens % (_NW * _CHUNK) == 0 and s % _CHUNK == 0

    ids_flat = input_ids.reshape(-1).astype(jnp.int32)
    fn = _make_sc_embed(n_tokens, s, hidden)
    out = fn(ids_flat, word_emb, pos_emb, type_emb)
    return out.reshape(b, s, hidden)


# speedup vs baseline: 2.0568x; 1.4888x over previous
import functools

import jax
import jax.numpy as jnp
from jax import lax
from jax.experimental import pallas as pl
from jax.experimental.pallas import tpu as pltpu
from jax.experimental.pallas import tpu_sc as plsc

_NC = 2
_NS = 16
_NW = _NC * _NS
_LANES = 16

_CHUNK = 32
_NROW = 3
_NPOS = 2


def _make_sc_embed(n_tokens, seq_len, hidden):
    per_w = n_tokens // _NW
    n_chunks = per_w // _CHUNK
    hgroups = hidden // _LANES

    mesh = plsc.VectorSubcoreMesh(core_axis_name="c", subcore_axis_name="s")

    @functools.partial(
        pl.kernel,
        mesh=mesh,
        out_type=jax.ShapeDtypeStruct((n_tokens, hidden), jnp.float32),
        scratch_types=(
            [pltpu.VMEM((per_w,), jnp.int32),
             pltpu.VMEM((hidden,), jnp.float32)]
            + [pltpu.VMEM((_CHUNK, hidden), jnp.float32) for _ in range(_NROW)]
            + [pltpu.VMEM((_CHUNK, hidden), jnp.float32) for _ in range(_NPOS)]
            + [pltpu.SemaphoreType.DMA for _ in range(2 * _NROW + _NPOS)]
        ),
    )
    def sc_embed(ids_hbm, wtab_hbm, ptab_hbm, ttab_hbm, out_hbm,
                 idx_v, type_v, *bufs_and_sems):
        rows = bufs_and_sems[:_NROW]
        pos = bufs_and_sems[_NROW:_NROW + _NPOS]
        gsem = bufs_and_sems[_NROW + _NPOS:2 * _NROW + _NPOS]
        osem = bufs_and_sems[2 * _NROW + _NPOS:3 * _NROW + _NPOS]
        psem = bufs_and_sems[3 * _NROW + _NPOS:]

        wid = lax.axis_index("s") * _NC + lax.axis_index("c")
        base = wid * per_w
        pltpu.sync_copy(ids_hbm.at[pl.ds(base, per_w)], idx_v)
        pltpu.sync_copy(ttab_hbm.at[0], type_v)

        def issue_gather(c):
            return pltpu.async_copy(
                wtab_hbm.at[idx_v.at[pl.ds(c * _CHUNK, _CHUNK)]],
                rows[c % _NROW], gsem[c % _NROW])

        def issue_pos(c):
            s0 = lax.rem(base + c * _CHUNK, seq_len)
            return pltpu.async_copy(
                ptab_hbm.at[pl.ds(s0, _CHUNK)], pos[c % _NPOS],
                psem[c % _NPOS])

        def issue_store(c):
            return pltpu.async_copy(
                rows[c % _NROW],
                out_hbm.at[pl.ds(base + c * _CHUNK, _CHUNK)],
                osem[c % _NROW])

        g = [None] * n_chunks
        p = [None] * n_chunks
        o = [None] * n_chunks
        for c in range(min(_NROW, n_chunks)):
            g[c] = issue_gather(c)
        for c in range(min(_NPOS, n_chunks)):
            p[c] = issue_pos(c)

        for c in range(n_chunks):
            if c >= 2:
                o[c - 2].wait()
                if c + 1 < n_chunks:
                    g[c + 1] = issue_gather(c + 1)
            g[c].wait()
            p[c].wait()
            rbuf = rows[c % _NROW]
            pbuf = pos[c % _NPOS]

            @plsc.parallel_loop(0, _CHUNK, unroll=1)
            def row_body(r):
                for h in range(hgroups):
                    sl = pl.ds(h * _LANES, _LANES)
                    plsc.addupdate(rbuf.at[r, sl], pbuf[r, sl] + type_v[sl])

            o[c] = issue_store(c)
            if c + _NPOS < n_chunks:
                p[c + _NPOS] = issue_pos(c + _NPOS)

        for c in range(max(0, n_chunks - 2), n_chunks):
            o[c].wait()

    return sc_embed


def kernel(input_ids, word_emb, pos_emb, type_emb):
    b, s = input_ids.shape
    hidden = word_emb.shape[1]
    n_tokens = b * s
    assert n_tokens % (_NW * _CHUNK) == 0 and s % _CHUNK == 0

    ids_flat = input_ids.reshape(-1).astype(jnp.int32)
    fn = _make_sc_embed(n_tokens, s, hidden)
    out = fn(ids_flat, word_emb, pos_emb, type_emb)
    return out.reshape(b, s, hidden)


# speedup vs baseline: 2.3531x; 1.1441x over previous
import functools

import jax
import jax.numpy as jnp
from jax import lax
from jax.experimental import pallas as pl
from jax.experimental.pallas import tpu as pltpu
from jax.experimental.pallas import tpu_sc as plsc

_NC = 2
_NS = 16
_NW = _NC * _NS
_LANES = 16

_CHUNK = 32
_NROW = 3


def _make_sc_embed(n_batch, seq_len, hidden):
    s_per_w = seq_len // _NW
    halves = s_per_w // _CHUNK
    n_chunks = n_batch * halves
    hgroups = hidden // _LANES
    n_tokens = n_batch * seq_len

    mesh = plsc.VectorSubcoreMesh(core_axis_name="c", subcore_axis_name="s")

    @functools.partial(
        pl.kernel,
        mesh=mesh,
        out_type=jax.ShapeDtypeStruct((n_tokens, hidden), jnp.float32),
        scratch_types=(
            [pltpu.VMEM((n_batch * s_per_w,), jnp.int32),
             pltpu.VMEM((hidden,), jnp.float32),
             pltpu.VMEM((s_per_w, hidden), jnp.float32)]
            + [pltpu.VMEM((_CHUNK, hidden), jnp.float32) for _ in range(_NROW)]
            + [pltpu.SemaphoreType.DMA for _ in range(2 * _NROW + 1)]
        ),
    )
    def sc_embed(ids_hbm, wtab_hbm, ptab_hbm, ttab_hbm, out_hbm,
                 idx_v, type_v, pos_v, *bufs_and_sems):
        rows = bufs_and_sems[:_NROW]
        gsem = bufs_and_sems[_NROW:2 * _NROW]
        osem = bufs_and_sems[2 * _NROW:3 * _NROW]
        psem = bufs_and_sems[3 * _NROW]

        wid = lax.axis_index("s") * _NC + lax.axis_index("c")
        s_base = wid * s_per_w
        for b in range(n_batch):
            pltpu.sync_copy(ids_hbm.at[pl.ds(b * seq_len + s_base, s_per_w)],
                            idx_v.at[pl.ds(b * s_per_w, s_per_w)])
        pos_cp = pltpu.async_copy(ptab_hbm.at[pl.ds(s_base, s_per_w)],
                                  pos_v, psem)
        pltpu.sync_copy(ttab_hbm.at[0], type_v)

        def issue_gather(c):
            return pltpu.async_copy(
                wtab_hbm.at[idx_v.at[pl.ds(c * _CHUNK, _CHUNK)]],
                rows[c % _NROW], gsem[c % _NROW])

        def issue_store(c):
            b, half = divmod(c, halves)
            off = b * seq_len + s_base + half * _CHUNK
            return pltpu.async_copy(
                rows[c % _NROW], out_hbm.at[pl.ds(off, _CHUNK)],
                osem[c % _NROW])

        g = [None] * n_chunks
        o = [None] * n_chunks
        for c in range(min(_NROW, n_chunks)):
            g[c] = issue_gather(c)

        pos_cp.wait()

        @plsc.parallel_loop(0, s_per_w, unroll=1)
        def pre_add(r):
            for h in range(hgroups):
                sl = pl.ds(h * _LANES, _LANES)
                plsc.addupdate(pos_v.at[r, sl], type_v[sl])

        for c in range(n_chunks):
            if c >= 2:
                o[c - 2].wait()
                if c + 1 < n_chunks:
                    g[c + 1] = issue_gather(c + 1)
            g[c].wait()
            rbuf = rows[c % _NROW]
            pbase = (c % halves) * _CHUNK

            @plsc.parallel_loop(0, _CHUNK, unroll=1)
            def row_body(r):
                for h in range(hgroups):
                    sl = pl.ds(h * _LANES, _LANES)
                    plsc.addupdate(rbuf.at[r, sl], pos_v[pbase + r, sl])

            o[c] = issue_store(c)

        for c in range(max(0, n_chunks - 2), n_chunks):
            o[c].wait()

    return sc_embed


def kernel(input_ids, word_emb, pos_emb, type_emb):
    b, s = input_ids.shape
    hidden = word_emb.shape[1]
    assert s % (_NW * _CHUNK) == 0

    ids_flat = input_ids.reshape(-1).astype(jnp.int32)
    fn = _make_sc_embed(b, s, hidden)
    out = fn(ids_flat, word_emb, pos_emb, type_emb)
    return out.reshape(b, s, hidden)
